# Initial kernel scaffold; baseline (speedup 1.0000x reference)
import jax
import jax.numpy as jnp
from jax.experimental import pallas as pl


def kernel(x, edge_index, batch, W, b):
    raise NotImplementedError("write your pallas kernel here")


# speedup vs baseline: 17.7880x; 17.7880x over previous
import functools

import jax
import jax.numpy as jnp
from jax.experimental import pallas as pl
from jax.experimental.pallas import tpu as pltpu

N = 100000
H = 128
G = 256
BLK = 2000
NB = N // BLK
NEG = -1e30


def _flash_body(x_ref, b3_ref, wt_ref, bias_ref, out_ref, m_run, s_run, acc):
    i = pl.program_id(0)

    @pl.when(i == 0)
    def _init():
        m_run[...] = jnp.full_like(m_run, NEG)
        s_run[...] = jnp.zeros_like(s_run)
        acc[...] = jnp.zeros_like(acc)

    x = x_ref[...]
    s = jnp.sum(x * wt_ref[...], axis=1) + bias_ref[0, 0]
    bb = b3_ref[0, 0, :]
    seg = jax.lax.broadcasted_iota(jnp.int32, (BLK, G), 1)
    oh = bb[:, None] == seg

    m_blk = jnp.max(jnp.where(oh, s[:, None], NEG), axis=0)
    m_new = jnp.maximum(m_run[0, :], m_blk)
    ratio = jnp.exp(m_run[0, :] - m_new)
    m_per_row = jnp.max(jnp.where(oh, m_new[None, :], NEG), axis=1)
    e = jnp.exp(s - m_per_row)
    ohf = oh.astype(jnp.float32)
    s_blk = jnp.sum(jnp.where(oh, e[:, None], 0.0), axis=0)
    s_run[...] = s_run[...] * ratio[None, :] + s_blk[None, :]
    m_run[...] = m_new[None, :]

    px = x * e[:, None]
    acc[...] = acc[...] * ratio[:, None] + jax.lax.dot_general(
        ohf, px, (((0,), (0,)), ((), ())),
        preferred_element_type=jnp.float32)

    @pl.when(i == NB - 1)
    def _fin():
        sr = s_run[0, :]
        out_ref[...] = acc[...] / (sr[:, None] + 1e-16)


def kernel(x, edge_index, batch, W, b):
    del edge_index
    wt = W.reshape(1, H)
    bias = b.reshape(1, 1)
    b3 = batch.reshape(NB, 1, BLK)
    gx = pl.pallas_call(
        _flash_body,
        grid=(NB,),
        in_specs=[
            pl.BlockSpec((BLK, H), lambda i: (i, 0)),
            pl.BlockSpec((1, 1, BLK), lambda i: (i, 0, 0)),
            pl.BlockSpec((1, H), lambda i: (0, 0)),
            pl.BlockSpec((1, 1), lambda i: (0, 0)),
        ],
        out_specs=pl.BlockSpec((G, H), lambda i: (0, 0)),
        out_shape=jax.ShapeDtypeStruct((G, H), jnp.float32),
        scratch_shapes=[
            pltpu.VMEM((1, G), jnp.float32),
            pltpu.VMEM((1, G), jnp.float32),
            pltpu.VMEM((G, H), jnp.float32),
        ],
    )(x, b3, wt, bias)
    return gx
